# Initial kernel scaffold; baseline (speedup 1.0000x reference)
import jax
import jax.numpy as jnp
from jax.experimental import pallas as pl


def kernel(node_features, edge_index, W1, b1, W2, b2):
    raise NotImplementedError("write your pallas kernel here")


# speedup vs baseline: 4.5112x; 4.5112x over previous
import functools

import jax
import jax.numpy as jnp
from jax import lax
from jax.experimental import pallas as pl
from jax.experimental.pallas import tpu as pltpu
from jax.experimental.pallas import tpu_sc as plsc

N = 50000
E = 800000
D_IN = 128
HID = 64
OUT = 32

L = 16
NC = 2
NS = 16

N_PAD = 50176
E_PAD = 802816
R = E_PAD // 128
STRIPE = N_PAD // NS
BN = 256
GRID_N = N_PAD // BN

_mesh = plsc.VectorSubcoreMesh(core_axis_name="c", subcore_axis_name="s")
_sc_params = pltpu.CompilerParams(use_tc_tiling_on_sc=False,
                                  needs_layout_passes=False)


def _deg_body(sd_h, part_h, deg_h, hist, idx_v, acc, tmp):
    c = lax.axis_index("c")
    s = lax.axis_index("s")

    zeros16 = jnp.zeros((L,), jnp.float32)

    def _zero(i, _):
        hist[pl.ds(i * L, L)] = zeros16
        return ()

    lax.fori_loop(0, N_PAD // L, _zero, (), unroll=4)

    rows_per_tile = R // NS
    base = s * rows_per_tile
    ones16 = jnp.full((L,), 1.0, jnp.float32)
    ki = 4

    def _hist_loop(it, _):
        r0 = base + it * ki
        pltpu.sync_copy(sd_h.at[c, pl.ds(r0, ki)], idx_v)
        for j in range(ki):
            for g in range(128 // L):
                iv = idx_v[j, pl.ds(g * L, L)]
                plsc.addupdate_scatter(hist, [iv], ones16)
        return ()

    lax.fori_loop(0, rows_per_tile // ki, _hist_loop, ())

    pltpu.sync_copy(hist, part_h.at[c, s])
    plsc.subcore_barrier()

    sl = pl.ds(s * STRIPE, STRIPE)
    pltpu.sync_copy(part_h.at[c, 0, sl], acc)

    def _accum(t, _):
        pltpu.sync_copy(part_h.at[c, t, sl], tmp)

        def _add(k, _):
            ksl = pl.ds(k * L, L)
            acc[ksl] = acc[ksl] + tmp[ksl]
            return ()

        lax.fori_loop(0, STRIPE // L, _add, (), unroll=4)
        return ()

    lax.fori_loop(1, NS, _accum, ())
    pltpu.sync_copy(acc, deg_h.at[c, sl])


_deg_kernel = functools.partial(
    pl.kernel,
    out_type=(
        jax.ShapeDtypeStruct((NC, NS, N_PAD), jnp.float32),
        jax.ShapeDtypeStruct((NC, N_PAD), jnp.float32),
    ),
    mesh=_mesh,
    scratch_types=[
        pltpu.VMEM((N_PAD,), jnp.float32),
        pltpu.VMEM((4, 128), jnp.int32),
        pltpu.VMEM((STRIPE,), jnp.float32),
        pltpu.VMEM((STRIPE,), jnp.float32),
    ],
    compiler_params=_sc_params,
)(_deg_body)


def _edge_body(tab_h, si_h, di_h, out_h, idx_s, idx_d, rows, zbuf, sh_acc):
    c = lax.axis_index("c")
    s = lax.axis_index("s")

    zeros16 = jnp.zeros((L,), jnp.float32)

    def _zero(i, _):
        for g in range(OUT // L):
            zbuf[i, pl.ds(g * L, L)] = zeros16
        return ()

    lax.fori_loop(0, zbuf.shape[0], _zero, (), unroll=4)

    zrows = zbuf.shape[0]
    for t in range(STRIPE // zrows):
        pltpu.sync_copy(zbuf, sh_acc.at[pl.ds(s * STRIPE + t * zrows, zrows)])
    plsc.subcore_barrier()

    rows_per_tile = R // NS
    base = s * rows_per_tile
    ki = 8

    def _edges(it, _):
        r0 = base + it * ki
        pltpu.sync_copy(si_h.at[c, pl.ds(r0, ki)], idx_s)
        pltpu.sync_copy(di_h.at[pl.ds(r0, ki)], idx_d)
        for j in range(ki):
            pltpu.sync_copy(tab_h.at[idx_s.at[j]], rows)
            pltpu.sync_copy(rows, sh_acc.at[idx_d.at[j]], add=True)
        return ()

    lax.fori_loop(0, rows_per_tile // ki, _edges, ())
    plsc.subcore_barrier()

    sl = pl.ds(s * STRIPE, STRIPE)
    pltpu.sync_copy(sh_acc.at[sl], out_h.at[c, sl])


_edge_kernel = functools.partial(
    pl.kernel,
    out_type=jax.ShapeDtypeStruct((NC, N_PAD, OUT), jnp.float32),
    mesh=_mesh,
    scratch_types=[
        pltpu.VMEM((8, 128), jnp.int32),
        pltpu.VMEM((8, 128), jnp.int32),
        pltpu.VMEM((128, OUT), jnp.float32),
        pltpu.VMEM((392, OUT), jnp.float32),
        pltpu.VMEM_SHARED((N_PAD, OUT), jnp.float32),
    ],
    compiler_params=_sc_params,
)(_edge_body)


def _rs(d):
    return lax.rsqrt(jnp.maximum(d, 1.0))


def _mm1_body(x_ref, do_ref, w1_ref, out_ref):
    xs = x_ref[...] * _rs(do_ref[...])
    h = jnp.dot(xs, w1_ref[...], preferred_element_type=jnp.float32)
    out_ref[0] = h[:, :OUT]
    out_ref[1] = h[:, OUT:]


def _mm2_body(agg_ref, do_ref, di_ref, w2_ref, b1_ref, out_ref):
    a = jnp.concatenate([agg_ref[0], agg_ref[1]], axis=1)
    t = jnp.maximum(a * _rs(di_ref[...]) + b1_ref[...], 0.0) * _rs(do_ref[...])
    h2 = jnp.dot(t, w2_ref[...], preferred_element_type=jnp.float32)
    out_ref[0] = h2
    out_ref[1] = h2


def _fin_body(p_ref, di_ref, b2_ref, out_ref):
    p = (p_ref[0] + p_ref[1]) * 0.5
    out_ref[...] = p * _rs(di_ref[...]) + b2_ref[...]


def _col_spec():
    return pl.BlockSpec((BN, 1), lambda i: (i, 0))


def _mm1(x_pad, do, W1):
    return pl.pallas_call(
        _mm1_body,
        grid=(GRID_N,),
        in_specs=[
            pl.BlockSpec((BN, D_IN), lambda i: (i, 0)),
            _col_spec(),
            pl.BlockSpec((D_IN, HID), lambda i: (0, 0)),
        ],
        out_specs=pl.BlockSpec((NC, BN, OUT), lambda i: (0, i, 0)),
        out_shape=jax.ShapeDtypeStruct((NC, N_PAD, OUT), jnp.float32),
    )(x_pad, do, W1)


def _mm2(agg, do, di, W2, b1):
    return pl.pallas_call(
        _mm2_body,
        grid=(GRID_N,),
        in_specs=[
            pl.BlockSpec((NC, BN, OUT), lambda i: (0, i, 0)),
            _col_spec(), _col_spec(),
            pl.BlockSpec((HID, OUT), lambda i: (0, 0)),
            pl.BlockSpec((1, HID), lambda i: (0, 0)),
        ],
        out_specs=pl.BlockSpec((NC, BN, OUT), lambda i: (0, i, 0)),
        out_shape=jax.ShapeDtypeStruct((NC, N_PAD, OUT), jnp.float32),
    )(agg, do, di, W2, b1)


def _fin(p, di, b2):
    return pl.pallas_call(
        _fin_body,
        grid=(GRID_N,),
        in_specs=[
            pl.BlockSpec((NC, BN, OUT), lambda i: (0, i, 0)),
            _col_spec(),
            pl.BlockSpec((1, OUT), lambda i: (0, 0)),
        ],
        out_specs=pl.BlockSpec((BN, OUT), lambda i: (i, 0)),
        out_shape=jax.ShapeDtypeStruct((N, OUT), jnp.float32),
    )(p, di, b2)


def kernel(node_features, edge_index, W1, b1, W2, b2):
    src = edge_index[0].astype(jnp.int32)
    dst = edge_index[1].astype(jnp.int32)
    pad = jnp.full((E_PAD - E,), N, jnp.int32)
    s2 = jnp.concatenate([src, pad]).reshape(R, 128)
    d2 = jnp.concatenate([dst, pad]).reshape(R, 128)
    sd = jnp.stack([s2, d2])
    s_stacked = jnp.stack([s2, s2 + N_PAD])

    x_pad = jnp.concatenate(
        [node_features, jnp.zeros((N_PAD - N, D_IN), jnp.float32)])

    _, deg = _deg_kernel(sd)
    do = deg[0].reshape(N_PAD, 1)
    di = deg[1].reshape(N_PAD, 1)

    h1 = _mm1(x_pad, do, W1)
    table0 = h1.reshape(NC * N_PAD, OUT)

    w1r = b1.reshape(1, HID)

    def _phase(table, _):
        agg = _edge_kernel(table, s_stacked, d2)
        nxt = _mm2(agg, do, di, W2, w1r).reshape(NC * N_PAD, OUT)
        return nxt, agg

    _, aggs = lax.scan(_phase, table0, None, length=2)
    return _fin(aggs[1], di, b2.reshape(1, OUT))


# speedup vs baseline: 5.8246x; 1.2912x over previous
import functools

import jax
import jax.numpy as jnp
from jax import lax
from jax.experimental import pallas as pl
from jax.experimental.pallas import tpu as pltpu
from jax.experimental.pallas import tpu_sc as plsc

N = 50000
E = 800000
D_IN = 128
HID = 64
OUT = 32

L = 16
NC = 2
NS = 16

N_PAD = 50176
E_PAD = 802816
R = E_PAD // 128
STRIPE = N_PAD // NS
BN = 256
GRID_N = N_PAD // BN

_mesh = plsc.VectorSubcoreMesh(core_axis_name="c", subcore_axis_name="s")
_sc_params = pltpu.CompilerParams(use_tc_tiling_on_sc=False,
                                  needs_layout_passes=False)


def _deg_body(sd_h, part_h, deg_h, hist, idx_v, acc, tmp):
    c = lax.axis_index("c")
    s = lax.axis_index("s")

    zeros16 = jnp.zeros((L,), jnp.float32)

    def _zero(i, _):
        hist[pl.ds(i * L, L)] = zeros16
        return ()

    lax.fori_loop(0, N_PAD // L, _zero, (), unroll=4)

    rows_per_tile = R // NS
    base = s * rows_per_tile
    ones16 = jnp.full((L,), 1.0, jnp.float32)
    ki = 4

    def _hist_loop(it, _):
        r0 = base + it * ki
        pltpu.sync_copy(sd_h.at[c, pl.ds(r0, ki)], idx_v)
        for j in range(ki):
            for g in range(128 // L):
                iv = idx_v[j, pl.ds(g * L, L)]
                plsc.addupdate_scatter(hist, [iv], ones16)
        return ()

    lax.fori_loop(0, rows_per_tile // ki, _hist_loop, ())

    pltpu.sync_copy(hist, part_h.at[c, s])
    plsc.subcore_barrier()

    sl = pl.ds(s * STRIPE, STRIPE)
    pltpu.sync_copy(part_h.at[c, 0, sl], acc)

    def _accum(t, _):
        pltpu.sync_copy(part_h.at[c, t, sl], tmp)

        def _add(k, _):
            ksl = pl.ds(k * L, L)
            acc[ksl] = acc[ksl] + tmp[ksl]
            return ()

        lax.fori_loop(0, STRIPE // L, _add, (), unroll=4)
        return ()

    lax.fori_loop(1, NS, _accum, ())
    pltpu.sync_copy(acc, deg_h.at[c, sl])


_deg_kernel = functools.partial(
    pl.kernel,
    out_type=(
        jax.ShapeDtypeStruct((NC, NS, N_PAD), jnp.float32),
        jax.ShapeDtypeStruct((NC, N_PAD), jnp.float32),
    ),
    mesh=_mesh,
    scratch_types=[
        pltpu.VMEM((N_PAD,), jnp.float32),
        pltpu.VMEM((4, 128), jnp.int32),
        pltpu.VMEM((STRIPE,), jnp.float32),
        pltpu.VMEM((STRIPE,), jnp.float32),
    ],
    compiler_params=_sc_params,
)(_deg_body)


KI = 2
BLK = KI * 128
NBLK = (R // NS) // KI
NPAIR = NBLK // 2
ZR = 196


def _edge_body(tab_h, si_h, di_h, out_h, idx_s, idx_d, rows0, rows1, zbuf,
               sh_acc, gs0, gs1, ss0, ss1):
    c = lax.axis_index("c")
    s = lax.axis_index("s")

    zeros16 = jnp.zeros((L,), jnp.float32)

    def _zero(i, _):
        for g in range(OUT // L):
            zbuf[i, pl.ds(g * L, L)] = zeros16
        return ()

    lax.fori_loop(0, zbuf.shape[0], _zero, (), unroll=4)

    zrows = zbuf.shape[0]
    for t in range(STRIPE // zrows):
        pltpu.sync_copy(zbuf, sh_acc.at[pl.ds(s * STRIPE + t * zrows, zrows)])
    plsc.subcore_barrier()

    base = s * (R // NS)

    def _load_idx(b, buf):
        r0 = base + b * KI
        pltpu.sync_copy(si_h.at[c, pl.ds(r0, KI)], idx_s.at[buf])
        pltpu.sync_copy(di_h.at[pl.ds(r0, KI)], idx_d.at[buf])

    def _fire_gathers(buf, rbuf, gsem):
        for j in range(KI):
            pltpu.async_copy(tab_h.at[idx_s.at[buf, j]],
                             rbuf.at[pl.ds(j * 128, 128)], gsem)

    def _fire_scatters(buf, rbuf, ssem):
        for j in range(KI):
            pltpu.async_copy(rbuf.at[pl.ds(j * 128, 128)],
                             sh_acc.at[idx_d.at[buf, j]], ssem, add=True)

    def _drain(sem, rbuf):
        pltpu.make_async_copy(tab_h.at[pl.ds(0, BLK)], rbuf, sem).wait()

    _load_idx(0, 0)
    _fire_gathers(0, rows0, gs0)

    def _pair(p, _):
        b0 = 2 * p
        _load_idx(b0 + 1, 1)

        @pl.when(p >= 1)
        def _():
            _drain(ss1, rows1)

        _fire_gathers(1, rows1, gs1)

        _drain(gs0, rows0)
        _fire_scatters(0, rows0, ss0)

        @pl.when(p + 1 < NPAIR)
        def _():
            _load_idx(b0 + 2, 0)
            _drain(ss0, rows0)
            _fire_gathers(0, rows0, gs0)

        _drain(gs1, rows1)
        _fire_scatters(1, rows1, ss1)
        return ()

    lax.fori_loop(0, NPAIR, _pair, ())
    _drain(ss0, rows0)
    _drain(ss1, rows1)
    plsc.subcore_barrier()

    sl = pl.ds(s * STRIPE, STRIPE)
    pltpu.sync_copy(sh_acc.at[sl], out_h.at[c, sl])


_edge_kernel = functools.partial(
    pl.kernel,
    out_type=jax.ShapeDtypeStruct((NC, N_PAD, OUT), jnp.float32),
    mesh=_mesh,
    scratch_types=[
        pltpu.VMEM((2, KI, 128), jnp.int32),
        pltpu.VMEM((2, KI, 128), jnp.int32),
        pltpu.VMEM((BLK, OUT), jnp.float32),
        pltpu.VMEM((BLK, OUT), jnp.float32),
        pltpu.VMEM((ZR, OUT), jnp.float32),
        pltpu.VMEM_SHARED((N_PAD, OUT), jnp.float32),
        pltpu.SemaphoreType.DMA,
        pltpu.SemaphoreType.DMA,
        pltpu.SemaphoreType.DMA,
        pltpu.SemaphoreType.DMA,
    ],
    compiler_params=_sc_params,
)(_edge_body)


def _rs(d):
    return lax.rsqrt(jnp.maximum(d, 1.0))


def _mm1_body(x_ref, do_ref, w1_ref, out_ref):
    xs = x_ref[...] * _rs(do_ref[...])
    h = jnp.dot(xs, w1_ref[...], preferred_element_type=jnp.float32)
    out_ref[0] = h[:, :OUT]
    out_ref[1] = h[:, OUT:]


def _mm2_body(agg_ref, do_ref, di_ref, w2_ref, b1_ref, out_ref):
    a = jnp.concatenate([agg_ref[0], agg_ref[1]], axis=1)
    t = jnp.maximum(a * _rs(di_ref[...]) + b1_ref[...], 0.0) * _rs(do_ref[...])
    h2 = jnp.dot(t, w2_ref[...], preferred_element_type=jnp.float32)
    out_ref[0] = h2
    out_ref[1] = h2


def _fin_body(p_ref, di_ref, b2_ref, out_ref):
    p = (p_ref[0] + p_ref[1]) * 0.5
    out_ref[...] = p * _rs(di_ref[...]) + b2_ref[...]


def _col_spec():
    return pl.BlockSpec((BN, 1), lambda i: (i, 0))


def _mm1(x_pad, do, W1):
    return pl.pallas_call(
        _mm1_body,
        grid=(GRID_N,),
        in_specs=[
            pl.BlockSpec((BN, D_IN), lambda i: (i, 0)),
            _col_spec(),
            pl.BlockSpec((D_IN, HID), lambda i: (0, 0)),
        ],
        out_specs=pl.BlockSpec((NC, BN, OUT), lambda i: (0, i, 0)),
        out_shape=jax.ShapeDtypeStruct((NC, N_PAD, OUT), jnp.float32),
    )(x_pad, do, W1)


def _mm2(agg, do, di, W2, b1):
    return pl.pallas_call(
        _mm2_body,
        grid=(GRID_N,),
        in_specs=[
            pl.BlockSpec((NC, BN, OUT), lambda i: (0, i, 0)),
            _col_spec(), _col_spec(),
            pl.BlockSpec((HID, OUT), lambda i: (0, 0)),
            pl.BlockSpec((1, HID), lambda i: (0, 0)),
        ],
        out_specs=pl.BlockSpec((NC, BN, OUT), lambda i: (0, i, 0)),
        out_shape=jax.ShapeDtypeStruct((NC, N_PAD, OUT), jnp.float32),
    )(agg, do, di, W2, b1)


def _fin(p, di, b2):
    return pl.pallas_call(
        _fin_body,
        grid=(GRID_N,),
        in_specs=[
            pl.BlockSpec((NC, BN, OUT), lambda i: (0, i, 0)),
            _col_spec(),
            pl.BlockSpec((1, OUT), lambda i: (0, 0)),
        ],
        out_specs=pl.BlockSpec((BN, OUT), lambda i: (i, 0)),
        out_shape=jax.ShapeDtypeStruct((N, OUT), jnp.float32),
    )(p, di, b2)


def kernel(node_features, edge_index, W1, b1, W2, b2):
    src = edge_index[0].astype(jnp.int32)
    dst = edge_index[1].astype(jnp.int32)
    pad = jnp.full((E_PAD - E,), N, jnp.int32)
    s2 = jnp.concatenate([src, pad]).reshape(R, 128)
    d2 = jnp.concatenate([dst, pad]).reshape(R, 128)
    sd = jnp.stack([s2, d2])
    s_stacked = jnp.stack([s2, s2 + N_PAD])

    x_pad = jnp.concatenate(
        [node_features, jnp.zeros((N_PAD - N, D_IN), jnp.float32)])

    _, deg = _deg_kernel(sd)
    do = deg[0].reshape(N_PAD, 1)
    di = deg[1].reshape(N_PAD, 1)

    h1 = _mm1(x_pad, do, W1)
    table0 = h1.reshape(NC * N_PAD, OUT)

    w1r = b1.reshape(1, HID)

    def _phase(table, _):
        agg = _edge_kernel(table, s_stacked, d2)
        nxt = _mm2(agg, do, di, W2, w1r).reshape(NC * N_PAD, OUT)
        return nxt, agg

    _, aggs = lax.scan(_phase, table0, None, length=2)
    return _fin(aggs[1], di, b2.reshape(1, OUT))


# speedup vs baseline: 7.8737x; 1.3518x over previous
import functools

import jax
import jax.numpy as jnp
from jax import lax
from jax.experimental import pallas as pl
from jax.experimental.pallas import tpu as pltpu
from jax.experimental.pallas import tpu_sc as plsc

N = 50000
E = 800000
D_IN = 128
HID = 64
OUT = 32

L = 16
NC = 2
NS = 16

N_PAD = 50176
E_PAD = 802816
R = E_PAD // 128
STRIPE = N_PAD // NS
BN = 1792
GRID_N = N_PAD // BN

_mesh = plsc.VectorSubcoreMesh(core_axis_name="c", subcore_axis_name="s")
_sc_params = pltpu.CompilerParams(use_tc_tiling_on_sc=False,
                                  needs_layout_passes=False)


def _deg_body(sd_h, part_h, deg_h, hist, idx_v, acc, tmp):
    c = lax.axis_index("c")
    s = lax.axis_index("s")

    zeros16 = jnp.zeros((L,), jnp.float32)

    def _zero(i, _):
        hist[pl.ds(i * L, L)] = zeros16
        return ()

    lax.fori_loop(0, N_PAD // L, _zero, (), unroll=4)

    rows_per_tile = R // NS
    base = s * rows_per_tile
    ones16 = jnp.full((L,), 1.0, jnp.float32)
    ki = 4

    def _hist_loop(it, _):
        r0 = base + it * ki
        pltpu.sync_copy(sd_h.at[c, pl.ds(r0, ki)], idx_v)
        for j in range(ki):
            for g in range(128 // L):
                iv = idx_v[j, pl.ds(g * L, L)]
                plsc.addupdate_scatter(hist, [iv], ones16)
        return ()

    lax.fori_loop(0, rows_per_tile // ki, _hist_loop, ())

    pltpu.sync_copy(hist, part_h.at[c, s])
    plsc.subcore_barrier()

    sl = pl.ds(s * STRIPE, STRIPE)
    pltpu.sync_copy(part_h.at[c, 0, sl], acc)

    def _accum(t, _):
        pltpu.sync_copy(part_h.at[c, t, sl], tmp)

        def _add(k, _):
            ksl = pl.ds(k * L, L)
            acc[ksl] = acc[ksl] + tmp[ksl]
            return ()

        lax.fori_loop(0, STRIPE // L, _add, (), unroll=4)
        return ()

    lax.fori_loop(1, NS, _accum, ())
    pltpu.sync_copy(acc, deg_h.at[c, sl])


_deg_kernel = functools.partial(
    pl.kernel,
    out_type=(
        jax.ShapeDtypeStruct((NC, NS, N_PAD), jnp.float32),
        jax.ShapeDtypeStruct((NC, N_PAD), jnp.float32),
    ),
    mesh=_mesh,
    scratch_types=[
        pltpu.VMEM((N_PAD,), jnp.float32),
        pltpu.VMEM((4, 128), jnp.int32),
        pltpu.VMEM((STRIPE,), jnp.float32),
        pltpu.VMEM((STRIPE,), jnp.float32),
    ],
    compiler_params=_sc_params,
)(_deg_body)


KI = 2
BLK = KI * 128
NBLK = (R // NS) // KI
NPAIR = NBLK // 2
ZR = 196


def _edge_body(tab_h, si_h, di_h, out_h, idx_s, idx_d, rows0, rows1, zbuf,
               sh_acc, gs0, gs1, ss0, ss1):
    c = lax.axis_index("c")
    s = lax.axis_index("s")

    zeros16 = jnp.zeros((L,), jnp.float32)

    def _zero(i, _):
        for g in range(OUT // L):
            zbuf[i, pl.ds(g * L, L)] = zeros16
        return ()

    lax.fori_loop(0, zbuf.shape[0], _zero, (), unroll=4)

    zrows = zbuf.shape[0]
    for t in range(STRIPE // zrows):
        pltpu.sync_copy(zbuf, sh_acc.at[pl.ds(s * STRIPE + t * zrows, zrows)])
    plsc.subcore_barrier()

    base = s * (R // NS)

    def _load_idx(b, buf):
        r0 = base + b * KI
        pltpu.sync_copy(si_h.at[c, pl.ds(r0, KI)], idx_s.at[buf])
        pltpu.sync_copy(di_h.at[pl.ds(r0, KI)], idx_d.at[buf])

    def _fire_gathers(buf, rbuf, gsem):
        for j in range(KI):
            pltpu.async_copy(tab_h.at[idx_s.at[buf, j]],
                             rbuf.at[pl.ds(j * 128, 128)], gsem)

    def _fire_scatters(buf, rbuf, ssem):
        for j in range(KI):
            pltpu.async_copy(rbuf.at[pl.ds(j * 128, 128)],
                             sh_acc.at[idx_d.at[buf, j]], ssem, add=True)

    def _drain(sem, rbuf):
        pltpu.make_async_copy(tab_h.at[pl.ds(0, BLK)], rbuf, sem).wait()

    _load_idx(0, 0)
    _fire_gathers(0, rows0, gs0)

    def _pair(p, _):
        b0 = 2 * p
        _load_idx(b0 + 1, 1)

        @pl.when(p >= 1)
        def _():
            _drain(ss1, rows1)

        _fire_gathers(1, rows1, gs1)

        _drain(gs0, rows0)
        _fire_scatters(0, rows0, ss0)

        @pl.when(p + 1 < NPAIR)
        def _():
            _load_idx(b0 + 2, 0)
            _drain(ss0, rows0)
            _fire_gathers(0, rows0, gs0)

        _drain(gs1, rows1)
        _fire_scatters(1, rows1, ss1)
        return ()

    lax.fori_loop(0, NPAIR, _pair, ())
    _drain(ss0, rows0)
    _drain(ss1, rows1)
    plsc.subcore_barrier()

    sl = pl.ds(s * STRIPE, STRIPE)
    pltpu.sync_copy(sh_acc.at[sl], out_h.at[c, sl])


_edge_kernel = functools.partial(
    pl.kernel,
    out_type=jax.ShapeDtypeStruct((NC, N_PAD, OUT), jnp.float32),
    mesh=_mesh,
    scratch_types=[
        pltpu.VMEM((2, KI, 128), jnp.int32),
        pltpu.VMEM((2, KI, 128), jnp.int32),
        pltpu.VMEM((BLK, OUT), jnp.float32),
        pltpu.VMEM((BLK, OUT), jnp.float32),
        pltpu.VMEM((ZR, OUT), jnp.float32),
        pltpu.VMEM_SHARED((N_PAD, OUT), jnp.float32),
        pltpu.SemaphoreType.DMA,
        pltpu.SemaphoreType.DMA,
        pltpu.SemaphoreType.DMA,
        pltpu.SemaphoreType.DMA,
    ],
    compiler_params=_sc_params,
)(_edge_body)


def _rs(d):
    return lax.rsqrt(jnp.maximum(d, 1.0))


def _mm1_body(x_ref, do_ref, w1_ref, out_ref):
    c = pl.program_id(0)
    xs = x_ref[...] * _rs(do_ref[...])
    h = jnp.dot(xs, w1_ref[...], preferred_element_type=jnp.float32)

    @pl.when(c == 0)
    def _():
        out_ref[...] = h[:, :OUT]

    @pl.when(c == 1)
    def _():
        out_ref[...] = h[:, OUT:]


def _mm2_body(agg_ref, do_ref, di_ref, w2_ref, b1_ref, out_ref):
    a = jnp.concatenate([agg_ref[0], agg_ref[1]], axis=1)
    t = jnp.maximum(a * _rs(di_ref[...]) + b1_ref[...], 0.0) * _rs(do_ref[...])
    out_ref[...] = jnp.dot(t, w2_ref[...], preferred_element_type=jnp.float32)


def _fin_body(p_ref, di_ref, b2_ref, out_ref):
    p = (p_ref[0] + p_ref[1]) * 0.5
    out_ref[...] = p * _rs(di_ref[...]) + b2_ref[...]


def _col_spec():
    return pl.BlockSpec((BN, 1), lambda c, i: (i, 0))


def _tab_spec():
    return pl.BlockSpec((BN, OUT), lambda c, i: (c * GRID_N + i, 0))


def _mm1(x_pad, do, W1):
    return pl.pallas_call(
        _mm1_body,
        grid=(NC, GRID_N),
        in_specs=[
            pl.BlockSpec((BN, D_IN), lambda c, i: (i, 0)),
            _col_spec(),
            pl.BlockSpec((D_IN, HID), lambda c, i: (0, 0)),
        ],
        out_specs=_tab_spec(),
        out_shape=jax.ShapeDtypeStruct((NC * N_PAD, OUT), jnp.float32),
    )(x_pad, do, W1)


def _mm2(agg, do, di, W2, b1):
    return pl.pallas_call(
        _mm2_body,
        grid=(NC, GRID_N),
        in_specs=[
            pl.BlockSpec((NC, BN, OUT), lambda c, i: (0, i, 0)),
            _col_spec(), _col_spec(),
            pl.BlockSpec((HID, OUT), lambda c, i: (0, 0)),
            pl.BlockSpec((1, HID), lambda c, i: (0, 0)),
        ],
        out_specs=_tab_spec(),
        out_shape=jax.ShapeDtypeStruct((NC * N_PAD, OUT), jnp.float32),
    )(agg, do, di, W2, b1)


def _fin(p, di, b2):
    return pl.pallas_call(
        _fin_body,
        grid=(GRID_N,),
        in_specs=[
            pl.BlockSpec((NC, BN, OUT), lambda i: (0, i, 0)),
            pl.BlockSpec((BN, 1), lambda i: (i, 0)),
            pl.BlockSpec((1, OUT), lambda i: (0, 0)),
        ],
        out_specs=pl.BlockSpec((BN, OUT), lambda i: (i, 0)),
        out_shape=jax.ShapeDtypeStruct((N, OUT), jnp.float32),
    )(p, di, b2)


def kernel(node_features, edge_index, W1, b1, W2, b2):
    src = edge_index[0].astype(jnp.int32)
    dst = edge_index[1].astype(jnp.int32)
    pad = jnp.full((E_PAD - E,), N, jnp.int32)
    s2 = jnp.concatenate([src, pad]).reshape(R, 128)
    d2 = jnp.concatenate([dst, pad]).reshape(R, 128)
    sd = jnp.stack([s2, d2])
    s_stacked = jnp.stack([s2, s2 + N_PAD])

    x_pad = jnp.concatenate(
        [node_features, jnp.zeros((N_PAD - N, D_IN), jnp.float32)])

    _, deg = _deg_kernel(sd)
    do = deg[0].reshape(N_PAD, 1)
    di = deg[1].reshape(N_PAD, 1)

    table0 = _mm1(x_pad, do, W1)

    w1r = b1.reshape(1, HID)

    def _phase(carry, _):
        table, _prev = carry
        agg = _edge_kernel(table, s_stacked, d2)
        nxt = _mm2(agg, do, di, W2, w1r)
        return (nxt, agg), None

    (_, agg2), _ = lax.scan(
        _phase, (table0, jnp.zeros((NC, N_PAD, OUT), jnp.float32)),
        None, length=2)
    return _fin(agg2, di, b2.reshape(1, OUT))


# speedup vs baseline: 9.2424x; 1.1738x over previous
import functools

import jax
import jax.numpy as jnp
from jax import lax
from jax.experimental import pallas as pl
from jax.experimental.pallas import tpu as pltpu
from jax.experimental.pallas import tpu_sc as plsc

N = 50000
E = 800000
D_IN = 128
HID = 64
OUT = 32

L = 16
NC = 2
NS = 16

N_PAD = 50176
E_PAD = 802816
R = E_PAD // 128
STRIPE = N_PAD // NS
BN = 1792
GRID_N = N_PAD // BN

_mesh = plsc.VectorSubcoreMesh(core_axis_name="c", subcore_axis_name="s")
_sc_params = pltpu.CompilerParams(use_tc_tiling_on_sc=False,
                                  needs_layout_passes=False)


SB = 28
NSB = (R // NS) // SB


def _deg_body(sd_h, part_h, deg_h, hist, idx0, idx1, acc, tmp0, tmp1,
              is0, is1, rs0, rs1):
    c = lax.axis_index("c")
    s = lax.axis_index("s")

    zeros16 = jnp.zeros((L,), jnp.float32)

    def _zero(i, _):
        hist[pl.ds(i * L, L)] = zeros16
        return ()

    lax.fori_loop(0, N_PAD // L, _zero, (), unroll=4)

    base = s * (R // NS)
    ones16 = jnp.full((L,), 1.0, jnp.float32)

    def _drain_idx(sem, buf):
        pltpu.make_async_copy(sd_h.at[c, pl.ds(0, SB)], buf, sem).wait()

    def _scatter_sb(buf):
        def _rows(j, _):
            for g in range(128 // L):
                iv = buf[j, pl.ds(g * L, L)]
                plsc.addupdate_scatter(hist, [iv], ones16)
            return ()

        lax.fori_loop(0, SB, _rows, (), unroll=4)

    pltpu.async_copy(sd_h.at[c, pl.ds(base, SB)], idx0, is0)

    def _sp(p, _):
        r0 = base + 2 * p * SB
        pltpu.async_copy(sd_h.at[c, pl.ds(r0 + SB, SB)], idx1, is1)
        _drain_idx(is0, idx0)
        _scatter_sb(idx0)

        @pl.when(p + 1 < NSB // 2)
        def _():
            pltpu.async_copy(sd_h.at[c, pl.ds(r0 + 2 * SB, SB)], idx0, is0)

        _drain_idx(is1, idx1)
        _scatter_sb(idx1)
        return ()

    lax.fori_loop(0, NSB // 2, _sp, ())

    pltpu.sync_copy(hist, part_h.at[c, s])
    plsc.subcore_barrier()

    sl = pl.ds(s * STRIPE, STRIPE)
    pltpu.sync_copy(part_h.at[c, 0, sl], acc)
    pltpu.async_copy(part_h.at[c, 1, sl], tmp0, rs0)

    def _drain_part(sem, buf):
        pltpu.make_async_copy(part_h.at[c, 0, sl], buf, sem).wait()

    for t in range(1, NS):
        buf, sem = (tmp0, rs0) if t % 2 == 1 else (tmp1, rs1)
        nbuf, nsem = (tmp1, rs1) if t % 2 == 1 else (tmp0, rs0)
        if t + 1 < NS:
            pltpu.async_copy(part_h.at[c, t + 1, sl], nbuf, nsem)
        _drain_part(sem, buf)

        def _add(k, _):
            ksl = pl.ds(k * L, L)
            acc[ksl] = acc[ksl] + buf[ksl]
            return ()

        lax.fori_loop(0, STRIPE // L, _add, (), unroll=4)

    pltpu.sync_copy(acc, deg_h.at[c, sl])


_deg_kernel = functools.partial(
    pl.kernel,
    out_type=(
        jax.ShapeDtypeStruct((NC, NS, N_PAD), jnp.float32),
        jax.ShapeDtypeStruct((NC, N_PAD), jnp.float32),
    ),
    mesh=_mesh,
    scratch_types=[
        pltpu.VMEM((N_PAD,), jnp.float32),
        pltpu.VMEM((SB, 128), jnp.int32),
        pltpu.VMEM((SB, 128), jnp.int32),
        pltpu.VMEM((STRIPE,), jnp.float32),
        pltpu.VMEM((STRIPE,), jnp.float32),
        pltpu.VMEM((STRIPE,), jnp.float32),
        pltpu.SemaphoreType.DMA,
        pltpu.SemaphoreType.DMA,
        pltpu.SemaphoreType.DMA,
        pltpu.SemaphoreType.DMA,
    ],
    compiler_params=_sc_params,
)(_deg_body)


KI = 2
BLK = KI * 128
NBLK = (R // NS) // KI
NPAIR = NBLK // 2
ZR = 196


def _edge_body(tab_h, si_h, di_h, par_h, out_h, idx_s, idx_d, rows0, rows1,
               zbuf, sh_acc, par_v, gs0, gs1, ss0, ss1):
    c = lax.axis_index("c")
    s = lax.axis_index("s")

    pltpu.sync_copy(par_h, par_v)
    pv = par_v[pl.ds(0, L)]
    lane = lax.iota(jnp.int32, L)
    rpt = jnp.sum(jnp.where(lane == 0, pv, 0))
    cstr = jnp.sum(jnp.where(lane == 1, pv, 0))

    zeros16 = jnp.zeros((L,), jnp.float32)

    def _zero(i, _):
        for g in range(OUT // L):
            zbuf[i, pl.ds(g * L, L)] = zeros16
        return ()

    lax.fori_loop(0, zbuf.shape[0], _zero, (), unroll=4)

    zrows = zbuf.shape[0]
    for t in range(STRIPE // zrows):
        pltpu.sync_copy(zbuf, sh_acc.at[pl.ds(s * STRIPE + t * zrows, zrows)])
    plsc.subcore_barrier()

    base = c * cstr + s * rpt
    npair = rpt // (2 * KI)

    def _load_idx(b, buf):
        r0 = base + b * KI
        pltpu.sync_copy(si_h.at[c, pl.ds(r0, KI)], idx_s.at[buf])
        pltpu.sync_copy(di_h.at[pl.ds(r0, KI)], idx_d.at[buf])

    def _fire_gathers(buf, rbuf, gsem):
        for j in range(KI):
            pltpu.async_copy(tab_h.at[idx_s.at[buf, j]],
                             rbuf.at[pl.ds(j * 128, 128)], gsem)

    def _fire_scatters(buf, rbuf, ssem):
        for j in range(KI):
            pltpu.async_copy(rbuf.at[pl.ds(j * 128, 128)],
                             sh_acc.at[idx_d.at[buf, j]], ssem, add=True)

    def _drain(sem, rbuf):
        pltpu.make_async_copy(tab_h.at[pl.ds(0, BLK)], rbuf, sem).wait()

    _load_idx(0, 0)
    _fire_gathers(0, rows0, gs0)

    def _pair(p, _):
        b0 = 2 * p
        _load_idx(b0 + 1, 1)

        @pl.when(p >= 1)
        def _():
            _drain(ss1, rows1)

        _fire_gathers(1, rows1, gs1)

        _drain(gs0, rows0)
        _fire_scatters(0, rows0, ss0)

        @pl.when(p + 1 < npair)
        def _():
            _load_idx(b0 + 2, 0)
            _drain(ss0, rows0)
            _fire_gathers(0, rows0, gs0)

        _drain(gs1, rows1)
        _fire_scatters(1, rows1, ss1)
        return ()

    lax.fori_loop(0, npair, _pair, ())
    _drain(ss0, rows0)
    _drain(ss1, rows1)
    plsc.subcore_barrier()

    sl = pl.ds(s * STRIPE, STRIPE)
    pltpu.sync_copy(sh_acc.at[sl], out_h.at[c, sl])


_edge_kernel = functools.partial(
    pl.kernel,
    out_type=jax.ShapeDtypeStruct((NC, N_PAD, OUT), jnp.float32),
    mesh=_mesh,
    scratch_types=[
        pltpu.VMEM((2, KI, 128), jnp.int32),
        pltpu.VMEM((2, KI, 128), jnp.int32),
        pltpu.VMEM((BLK, OUT), jnp.float32),
        pltpu.VMEM((BLK, OUT), jnp.float32),
        pltpu.VMEM((ZR, OUT), jnp.float32),
        pltpu.VMEM_SHARED((N_PAD, OUT), jnp.float32),
        pltpu.VMEM((L,), jnp.int32),
        pltpu.SemaphoreType.DMA,
        pltpu.SemaphoreType.DMA,
        pltpu.SemaphoreType.DMA,
        pltpu.SemaphoreType.DMA,
    ],
    compiler_params=_sc_params,
)(_edge_body)


def _rs(d):
    return lax.rsqrt(jnp.maximum(d, 1.0))


def _mm1_body(x_ref, do_ref, w1_ref, out_ref):
    c = pl.program_id(0)
    xs = x_ref[...] * _rs(do_ref[...])
    h = jnp.dot(xs, w1_ref[...], preferred_element_type=jnp.float32)

    @pl.when(c == 0)
    def _():
        out_ref[...] = h[:, :OUT]

    @pl.when(c == 1)
    def _():
        out_ref[...] = h[:, OUT:]


def _mm2_body(agg_ref, do_ref, di_ref, w2_ref, b1_ref, out_ref):
    a = jnp.concatenate([agg_ref[0], agg_ref[1]], axis=1)
    t = jnp.maximum(a * _rs(di_ref[...]) + b1_ref[...], 0.0) * _rs(do_ref[...])
    out_ref[...] = jnp.dot(t, w2_ref[...], preferred_element_type=jnp.float32)


def _fin_body(p_ref, di_ref, b2_ref, out_ref):
    p = p_ref[0] + p_ref[1]
    out_ref[...] = p * _rs(di_ref[...]) + b2_ref[...]


def _col_spec():
    return pl.BlockSpec((BN, 1), lambda c, i: (i, 0))


def _tab_spec():
    return pl.BlockSpec((BN, OUT), lambda c, i: (c * GRID_N + i, 0))


def _mm1(x_pad, do, W1):
    return pl.pallas_call(
        _mm1_body,
        grid=(NC, GRID_N),
        in_specs=[
            pl.BlockSpec((BN, D_IN), lambda c, i: (i, 0)),
            _col_spec(),
            pl.BlockSpec((D_IN, HID), lambda c, i: (0, 0)),
        ],
        out_specs=_tab_spec(),
        out_shape=jax.ShapeDtypeStruct((NC * N_PAD, OUT), jnp.float32),
    )(x_pad, do, W1)


def _mm2(agg, do, di, W2, b1):
    return pl.pallas_call(
        _mm2_body,
        grid=(NC, GRID_N),
        in_specs=[
            pl.BlockSpec((NC, BN, OUT), lambda c, i: (0, i, 0)),
            _col_spec(), _col_spec(),
            pl.BlockSpec((HID, OUT), lambda c, i: (0, 0)),
            pl.BlockSpec((1, HID), lambda c, i: (0, 0)),
        ],
        out_specs=_tab_spec(),
        out_shape=jax.ShapeDtypeStruct((NC * N_PAD, OUT), jnp.float32),
    )(agg, do, di, W2, b1)


def _fin(p, di, b2):
    return pl.pallas_call(
        _fin_body,
        grid=(GRID_N,),
        in_specs=[
            pl.BlockSpec((NC, BN, OUT), lambda i: (0, i, 0)),
            pl.BlockSpec((BN, 1), lambda i: (i, 0)),
            pl.BlockSpec((1, OUT), lambda i: (0, 0)),
        ],
        out_specs=pl.BlockSpec((BN, OUT), lambda i: (i, 0)),
        out_shape=jax.ShapeDtypeStruct((N, OUT), jnp.float32),
    )(p, di, b2)


def kernel(node_features, edge_index, W1, b1, W2, b2):
    src = edge_index[0].astype(jnp.int32)
    dst = edge_index[1].astype(jnp.int32)
    pad = jnp.full((E_PAD - E,), N, jnp.int32)
    s2 = jnp.concatenate([src, pad]).reshape(R, 128)
    d2 = jnp.concatenate([dst, pad]).reshape(R, 128)
    sd = jnp.stack([s2, d2])
    s_stacked = jnp.stack([s2, s2 + N_PAD])

    x_pad = jnp.concatenate(
        [node_features, jnp.zeros((N_PAD - N, D_IN), jnp.float32)])

    _, deg = _deg_kernel(sd)
    do = deg[0].reshape(N_PAD, 1)
    di = deg[1].reshape(N_PAD, 1)

    table0 = _mm1(x_pad, do, W1)

    w1r = b1.reshape(1, HID)

    params = jnp.zeros((2, L), jnp.int32)
    params = params.at[0, 0].set(R // NS)
    params = params.at[1, 0].set(R // (NC * NS))
    params = params.at[1, 1].set(R // NC)

    def _phase(carry, par):
        table, _prev = carry
        agg = _edge_kernel(table, s_stacked, d2, par)
        nxt = _mm2(agg, do, di, W2, w1r)
        return (nxt, agg), None

    (_, agg2), _ = lax.scan(
        _phase, (table0, jnp.zeros((NC, N_PAD, OUT), jnp.float32)),
        params, length=2)
    return _fin(agg2, di, b2.reshape(1, OUT))


# speedup vs baseline: 10.6224x; 1.1493x over previous
import functools

import jax
import jax.numpy as jnp
from jax import lax
from jax.experimental import pallas as pl
from jax.experimental.pallas import tpu as pltpu
from jax.experimental.pallas import tpu_sc as plsc

N = 50000
E = 800000
D_IN = 128
HID = 64
OUT = 32

L = 16
NC = 2
NS = 16

N_PAD = 50176
E_PAD = 802816
R = E_PAD // 128
STRIPE = N_PAD // NS
BN = 1792
GRID_N = N_PAD // BN

_mesh = plsc.VectorSubcoreMesh(core_axis_name="c", subcore_axis_name="s")
_sc_params = pltpu.CompilerParams(use_tc_tiling_on_sc=False,
                                  needs_layout_passes=False)


SB = 28
NSB = (R // NS) // SB


def _deg_body(sd_h, part_h, deg_h, hist, idx0, idx1, acc, tmp0, tmp1,
              is0, is1, rs0, rs1):
    c = lax.axis_index("c")
    s = lax.axis_index("s")

    zeros16 = jnp.zeros((L,), jnp.float32)

    def _zero(i, _):
        hist[pl.ds(i * L, L)] = zeros16
        return ()

    lax.fori_loop(0, N_PAD // L, _zero, (), unroll=4)

    base = s * (R // NS)
    ones16 = jnp.full((L,), 1.0, jnp.float32)

    def _drain_idx(sem, buf):
        pltpu.make_async_copy(sd_h.at[c, pl.ds(0, SB)], buf, sem).wait()

    def _scatter_sb(buf):
        def _rows(j, _):
            for g in range(128 // L):
                iv = buf[j, pl.ds(g * L, L)]
                plsc.addupdate_scatter(hist, [iv], ones16)
            return ()

        lax.fori_loop(0, SB, _rows, (), unroll=4)

    pltpu.async_copy(sd_h.at[c, pl.ds(base, SB)], idx0, is0)

    def _sp(p, _):
        r0 = base + 2 * p * SB
        pltpu.async_copy(sd_h.at[c, pl.ds(r0 + SB, SB)], idx1, is1)
        _drain_idx(is0, idx0)
        _scatter_sb(idx0)

        @pl.when(p + 1 < NSB // 2)
        def _():
            pltpu.async_copy(sd_h.at[c, pl.ds(r0 + 2 * SB, SB)], idx0, is0)

        _drain_idx(is1, idx1)
        _scatter_sb(idx1)
        return ()

    lax.fori_loop(0, NSB // 2, _sp, ())

    pltpu.sync_copy(hist, part_h.at[c, s])
    plsc.subcore_barrier()

    sl = pl.ds(s * STRIPE, STRIPE)
    pltpu.sync_copy(part_h.at[c, 0, sl], acc)
    pltpu.async_copy(part_h.at[c, 1, sl], tmp0, rs0)

    def _drain_part(sem, buf):
        pltpu.make_async_copy(part_h.at[c, 0, sl], buf, sem).wait()

    for t in range(1, NS):
        buf, sem = (tmp0, rs0) if t % 2 == 1 else (tmp1, rs1)
        nbuf, nsem = (tmp1, rs1) if t % 2 == 1 else (tmp0, rs0)
        if t + 1 < NS:
            pltpu.async_copy(part_h.at[c, t + 1, sl], nbuf, nsem)
        _drain_part(sem, buf)

        def _add(k, _):
            ksl = pl.ds(k * L, L)
            acc[ksl] = acc[ksl] + buf[ksl]
            return ()

        lax.fori_loop(0, STRIPE // L, _add, (), unroll=4)

    pltpu.sync_copy(acc, deg_h.at[c, sl])


_deg_kernel = functools.partial(
    pl.kernel,
    out_type=(
        jax.ShapeDtypeStruct((NC, NS, N_PAD), jnp.float32),
        jax.ShapeDtypeStruct((NC, N_PAD), jnp.float32),
    ),
    mesh=_mesh,
    scratch_types=[
        pltpu.VMEM((N_PAD,), jnp.float32),
        pltpu.VMEM((SB, 128), jnp.int32),
        pltpu.VMEM((SB, 128), jnp.int32),
        pltpu.VMEM((STRIPE,), jnp.float32),
        pltpu.VMEM((STRIPE,), jnp.float32),
        pltpu.VMEM((STRIPE,), jnp.float32),
        pltpu.SemaphoreType.DMA,
        pltpu.SemaphoreType.DMA,
        pltpu.SemaphoreType.DMA,
        pltpu.SemaphoreType.DMA,
    ],
    compiler_params=_sc_params,
)(_deg_body)


KI = 2
BLK = KI * 128
NBLK = (R // NS) // KI
NPAIR = NBLK // 2
ZR = 196


def _edge_body(tab_h, si_h, di_h, par_h, out_h, idx_s, idx_d, rows0, rows1,
               zbuf, sh_acc, par_v, gs0, gs1, ss0, ss1):
    c = lax.axis_index("c")
    s = lax.axis_index("s")

    pltpu.sync_copy(par_h, par_v)
    pv = par_v[pl.ds(0, L)]
    lane = lax.iota(jnp.int32, L)
    rpt = jnp.sum(jnp.where(lane == 0, pv, 0))
    cstr = jnp.sum(jnp.where(lane == 1, pv, 0))

    zeros16 = jnp.zeros((L,), jnp.float32)

    def _zero(i, _):
        for g in range(OUT // L):
            zbuf[i, pl.ds(g * L, L)] = zeros16
        return ()

    lax.fori_loop(0, zbuf.shape[0], _zero, (), unroll=4)

    zrows = zbuf.shape[0]
    for t in range(STRIPE // zrows):
        pltpu.sync_copy(zbuf, sh_acc.at[pl.ds(s * STRIPE + t * zrows, zrows)])
    plsc.subcore_barrier()

    base = c * cstr + s * rpt
    npair = rpt // (2 * KI)
    off = (c * N_PAD).astype(jnp.int32)

    def _load_idx(b, buf):
        r0 = base + b * KI
        pltpu.sync_copy(si_h.at[pl.ds(r0, KI)], idx_s.at[buf])
        pltpu.sync_copy(di_h.at[pl.ds(r0, KI)], idx_d.at[buf])
        for j in range(KI):
            for g in range(128 // L):
                gsl = pl.ds(g * L, L)
                idx_s[buf, j, gsl] = idx_s[buf, j, gsl] + off

    def _fire_gathers(buf, rbuf, gsem):
        for j in range(KI):
            pltpu.async_copy(tab_h.at[idx_s.at[buf, j]],
                             rbuf.at[pl.ds(j * 128, 128)], gsem)

    def _fire_scatters(buf, rbuf, ssem):
        for j in range(KI):
            pltpu.async_copy(rbuf.at[pl.ds(j * 128, 128)],
                             sh_acc.at[idx_d.at[buf, j]], ssem, add=True)

    def _drain(sem, rbuf):
        pltpu.make_async_copy(tab_h.at[pl.ds(0, BLK)], rbuf, sem).wait()

    _load_idx(0, 0)
    _fire_gathers(0, rows0, gs0)

    def _pair(p, _):
        b0 = 2 * p
        _load_idx(b0 + 1, 1)

        @pl.when(p >= 1)
        def _():
            _drain(ss1, rows1)

        _fire_gathers(1, rows1, gs1)

        _drain(gs0, rows0)
        _fire_scatters(0, rows0, ss0)

        @pl.when(p + 1 < npair)
        def _():
            _load_idx(b0 + 2, 0)
            _drain(ss0, rows0)
            _fire_gathers(0, rows0, gs0)

        _drain(gs1, rows1)
        _fire_scatters(1, rows1, ss1)
        return ()

    lax.fori_loop(0, npair, _pair, ())
    _drain(ss0, rows0)
    _drain(ss1, rows1)
    plsc.subcore_barrier()

    sl = pl.ds(s * STRIPE, STRIPE)
    pltpu.sync_copy(sh_acc.at[sl], out_h.at[c, sl])


_edge_kernel = functools.partial(
    pl.kernel,
    out_type=jax.ShapeDtypeStruct((NC, N_PAD, OUT), jnp.float32),
    mesh=_mesh,
    scratch_types=[
        pltpu.VMEM((2, KI, 128), jnp.int32),
        pltpu.VMEM((2, KI, 128), jnp.int32),
        pltpu.VMEM((BLK, OUT), jnp.float32),
        pltpu.VMEM((BLK, OUT), jnp.float32),
        pltpu.VMEM((ZR, OUT), jnp.float32),
        pltpu.VMEM_SHARED((N_PAD, OUT), jnp.float32),
        pltpu.VMEM((L,), jnp.int32),
        pltpu.SemaphoreType.DMA,
        pltpu.SemaphoreType.DMA,
        pltpu.SemaphoreType.DMA,
        pltpu.SemaphoreType.DMA,
    ],
    compiler_params=_sc_params,
)(_edge_body)


def _rs(d):
    return lax.rsqrt(jnp.maximum(d, 1.0))


def _mm1_body(x_ref, d_ref, w1_ref, out_ref):
    xs = x_ref[...] * _rs(d_ref[:, 0:1])
    h = jnp.dot(xs, w1_ref[...], preferred_element_type=jnp.float32)
    out_ref[0] = h[:, :OUT]
    out_ref[1] = h[:, OUT:]


def _mm2_body(agg_ref, d_ref, w2_ref, b1_ref, out_ref):
    a = jnp.concatenate([agg_ref[0], agg_ref[1]], axis=1)
    t = (jnp.maximum(a * _rs(d_ref[:, 1:2]) + b1_ref[...], 0.0)
         * _rs(d_ref[:, 0:1]))
    h2 = jnp.dot(t, w2_ref[...], preferred_element_type=jnp.float32)
    out_ref[0] = h2
    out_ref[1] = h2


def _fin_body(p_ref, d_ref, b2_ref, out_ref):
    p = p_ref[0] + p_ref[1]
    out_ref[...] = p * _rs(d_ref[:, 1:2]) + b2_ref[...]


def _dcol_spec():
    return pl.BlockSpec((BN, 2), lambda i: (i, 0))


def _mm1(x_pad, dcol, W1):
    return pl.pallas_call(
        _mm1_body,
        grid=(GRID_N,),
        in_specs=[
            pl.BlockSpec((BN, D_IN), lambda i: (i, 0)),
            _dcol_spec(),
            pl.BlockSpec((D_IN, HID), lambda i: (0, 0)),
        ],
        out_specs=pl.BlockSpec((NC, BN, OUT), lambda i: (0, i, 0)),
        out_shape=jax.ShapeDtypeStruct((NC, N_PAD, OUT), jnp.float32),
    )(x_pad, dcol, W1)


def _mm2(agg, dcol, W2, b1):
    return pl.pallas_call(
        _mm2_body,
        grid=(GRID_N,),
        in_specs=[
            pl.BlockSpec((NC, BN, OUT), lambda i: (0, i, 0)),
            _dcol_spec(),
            pl.BlockSpec((HID, OUT), lambda i: (0, 0)),
            pl.BlockSpec((1, HID), lambda i: (0, 0)),
        ],
        out_specs=pl.BlockSpec((NC, BN, OUT), lambda i: (0, i, 0)),
        out_shape=jax.ShapeDtypeStruct((NC, N_PAD, OUT), jnp.float32),
    )(agg, dcol, W2, b1)


def _fin(p, dcol, b2):
    return pl.pallas_call(
        _fin_body,
        grid=(GRID_N,),
        in_specs=[
            pl.BlockSpec((NC, BN, OUT), lambda i: (0, i, 0)),
            _dcol_spec(),
            pl.BlockSpec((1, OUT), lambda i: (0, 0)),
        ],
        out_specs=pl.BlockSpec((BN, OUT), lambda i: (i, 0)),
        out_shape=jax.ShapeDtypeStruct((N, OUT), jnp.float32),
    )(p, dcol, b2)


def kernel(node_features, edge_index, W1, b1, W2, b2):
    src = edge_index[0].astype(jnp.int32)
    dst = edge_index[1].astype(jnp.int32)
    pad = jnp.full((E_PAD - E,), N, jnp.int32)
    s2 = jnp.concatenate([src, pad]).reshape(R, 128)
    d2 = jnp.concatenate([dst, pad]).reshape(R, 128)
    sd = jnp.stack([s2, d2])

    x_pad = jnp.concatenate(
        [node_features, jnp.zeros((N_PAD - N, D_IN), jnp.float32)])

    _, deg = _deg_kernel(sd)
    dcol = jnp.stack([deg[0], deg[1]], axis=1)

    table0 = _mm1(x_pad, dcol, W1).reshape(NC * N_PAD, OUT)

    w1r = b1.reshape(1, HID)

    params = jnp.zeros((2, L), jnp.int32)
    params = params.at[0, 0].set(R // NS)
    params = params.at[1, 0].set(R // (NC * NS))
    params = params.at[1, 1].set(R // NC)

    def _phase(carry, par):
        table, _prev = carry
        agg = _edge_kernel(table, s2, d2, par)
        nxt = _mm2(agg, dcol, W2, w1r).reshape(NC * N_PAD, OUT)
        return (nxt, agg), None

    (_, agg2), _ = lax.scan(
        _phase, (table0, jnp.zeros((NC, N_PAD, OUT), jnp.float32)),
        params, length=2)
    return _fin(agg2, dcol, b2.reshape(1, OUT))


# speedup vs baseline: 10.7237x; 1.0095x over previous
import functools

import jax
import jax.numpy as jnp
from jax import lax
from jax.experimental import pallas as pl
from jax.experimental.pallas import tpu as pltpu
from jax.experimental.pallas import tpu_sc as plsc

N = 50000
E = 800000
D_IN = 128
HID = 64
OUT = 32

L = 16
NC = 2
NS = 16

N_PAD = 50176
E_PAD = 802816
R = E_PAD // 128
STRIPE = N_PAD // NS
BN = 1792
GRID_N = N_PAD // BN

_mesh = plsc.VectorSubcoreMesh(core_axis_name="c", subcore_axis_name="s")
_sc_params = pltpu.CompilerParams(use_tc_tiling_on_sc=False,
                                  needs_layout_passes=False)


SB = 28
NSB = (R // NS) // SB


def _deg_body(si_h, di_h, part_h, deg_h, hist, idx0, idx1, acc, tmp0, tmp1,
              is0, is1, rs0, rs1):
    c = lax.axis_index("c")
    s = lax.axis_index("s")

    zeros16 = jnp.zeros((L,), jnp.float32)

    def _zero(i, _):
        hist[pl.ds(i * L, L)] = zeros16
        return ()

    lax.fori_loop(0, N_PAD // L, _zero, (), unroll=4)

    base = s * (R // NS)
    ones16 = jnp.full((L,), 1.0, jnp.float32)

    def _fetch_idx(rows, buf, sem):
        @pl.when(c == 0)
        def _():
            pltpu.async_copy(si_h.at[rows], buf, sem)

        @pl.when(c == 1)
        def _():
            pltpu.async_copy(di_h.at[rows], buf, sem)

    def _drain_idx(sem, buf):
        pltpu.make_async_copy(si_h.at[pl.ds(0, SB)], buf, sem).wait()

    def _scatter_sb(buf):
        def _rows(j, _):
            for g in range(128 // L):
                iv = buf[j, pl.ds(g * L, L)]
                plsc.addupdate_scatter(hist, [iv], ones16)
            return ()

        lax.fori_loop(0, SB, _rows, (), unroll=4)

    _fetch_idx(pl.ds(base, SB), idx0, is0)

    def _sp(p, _):
        r0 = base + 2 * p * SB
        _fetch_idx(pl.ds(r0 + SB, SB), idx1, is1)
        _drain_idx(is0, idx0)
        _scatter_sb(idx0)

        @pl.when(p + 1 < NSB // 2)
        def _():
            _fetch_idx(pl.ds(r0 + 2 * SB, SB), idx0, is0)

        _drain_idx(is1, idx1)
        _scatter_sb(idx1)
        return ()

    lax.fori_loop(0, NSB // 2, _sp, ())

    pltpu.sync_copy(hist, part_h.at[c, s])
    plsc.subcore_barrier()

    sl = pl.ds(s * STRIPE, STRIPE)
    pltpu.sync_copy(part_h.at[c, 0, sl], acc)
    pltpu.async_copy(part_h.at[c, 1, sl], tmp0, rs0)

    def _drain_part(sem, buf):
        pltpu.make_async_copy(part_h.at[c, 0, sl], buf, sem).wait()

    for t in range(1, NS):
        buf, sem = (tmp0, rs0) if t % 2 == 1 else (tmp1, rs1)
        nbuf, nsem = (tmp1, rs1) if t % 2 == 1 else (tmp0, rs0)
        if t + 1 < NS:
            pltpu.async_copy(part_h.at[c, t + 1, sl], nbuf, nsem)
        _drain_part(sem, buf)

        def _add(k, _):
            ksl = pl.ds(k * L, L)
            acc[ksl] = acc[ksl] + buf[ksl]
            return ()

        lax.fori_loop(0, STRIPE // L, _add, (), unroll=4)

    pltpu.sync_copy(acc, deg_h.at[c, sl])


_deg_kernel = functools.partial(
    pl.kernel,
    out_type=(
        jax.ShapeDtypeStruct((NC, NS, N_PAD), jnp.float32),
        jax.ShapeDtypeStruct((NC, N_PAD), jnp.float32),
    ),
    mesh=_mesh,
    scratch_types=[
        pltpu.VMEM((N_PAD,), jnp.float32),
        pltpu.VMEM((SB, 128), jnp.int32),
        pltpu.VMEM((SB, 128), jnp.int32),
        pltpu.VMEM((STRIPE,), jnp.float32),
        pltpu.VMEM((STRIPE,), jnp.float32),
        pltpu.VMEM((STRIPE,), jnp.float32),
        pltpu.SemaphoreType.DMA,
        pltpu.SemaphoreType.DMA,
        pltpu.SemaphoreType.DMA,
        pltpu.SemaphoreType.DMA,
    ],
    compiler_params=_sc_params,
)(_deg_body)


KI = 2
BLK = KI * 128
NBLK = (R // NS) // KI
NPAIR = NBLK // 2
ZR = 196


def _edge_body(tab_h, si_h, di_h, par_h, out_h, idx_s, idx_d, rows0, rows1,
               zbuf, sh_acc, par_v, gs0, gs1, ss0, ss1):
    c = lax.axis_index("c")
    s = lax.axis_index("s")

    pltpu.sync_copy(par_h, par_v)
    pv = par_v[pl.ds(0, L)]
    lane = lax.iota(jnp.int32, L)
    rpt = jnp.sum(jnp.where(lane == 0, pv, 0))
    cstr = jnp.sum(jnp.where(lane == 1, pv, 0))
    goff = jnp.sum(jnp.where(lane == 2, pv, 0))

    zeros16 = jnp.zeros((L,), jnp.float32)

    def _zero(i, _):
        for g in range(OUT // L):
            zbuf[i, pl.ds(g * L, L)] = zeros16
        return ()

    lax.fori_loop(0, zbuf.shape[0], _zero, (), unroll=4)

    zrows = zbuf.shape[0]
    for t in range(STRIPE // zrows):
        pltpu.sync_copy(zbuf, sh_acc.at[pl.ds(s * STRIPE + t * zrows, zrows)])
    plsc.subcore_barrier()

    base = c * cstr + s * rpt
    npair = rpt // (2 * KI)
    off = (c * goff).astype(jnp.int32)

    def _load_idx(b, buf):
        r0 = base + b * KI
        pltpu.sync_copy(si_h.at[pl.ds(r0, KI)], idx_s.at[buf])
        pltpu.sync_copy(di_h.at[pl.ds(r0, KI)], idx_d.at[buf])
        for j in range(KI):
            for g in range(128 // L):
                gsl = pl.ds(g * L, L)
                idx_s[buf, j, gsl] = idx_s[buf, j, gsl] + off

    def _fire_gathers(buf, rbuf, gsem):
        for j in range(KI):
            pltpu.async_copy(tab_h.at[idx_s.at[buf, j]],
                             rbuf.at[pl.ds(j * 128, 128)], gsem)

    def _fire_scatters(buf, rbuf, ssem):
        for j in range(KI):
            pltpu.async_copy(rbuf.at[pl.ds(j * 128, 128)],
                             sh_acc.at[idx_d.at[buf, j]], ssem, add=True)

    def _drain(sem, rbuf):
        pltpu.make_async_copy(tab_h.at[pl.ds(0, BLK)], rbuf, sem).wait()

    _load_idx(0, 0)
    _fire_gathers(0, rows0, gs0)

    def _pair(p, _):
        b0 = 2 * p
        _load_idx(b0 + 1, 1)

        @pl.when(p >= 1)
        def _():
            _drain(ss1, rows1)

        _fire_gathers(1, rows1, gs1)

        _drain(gs0, rows0)
        _fire_scatters(0, rows0, ss0)

        @pl.when(p + 1 < npair)
        def _():
            _load_idx(b0 + 2, 0)
            _drain(ss0, rows0)
            _fire_gathers(0, rows0, gs0)

        _drain(gs1, rows1)
        _fire_scatters(1, rows1, ss1)
        return ()

    lax.fori_loop(0, npair, _pair, ())
    _drain(ss0, rows0)
    _drain(ss1, rows1)
    plsc.subcore_barrier()

    sl = pl.ds(s * STRIPE, STRIPE)
    pltpu.sync_copy(sh_acc.at[sl], out_h.at[c, sl])


_edge_kernel = functools.partial(
    pl.kernel,
    out_type=jax.ShapeDtypeStruct((NC, N_PAD, OUT), jnp.float32),
    mesh=_mesh,
    scratch_types=[
        pltpu.VMEM((2, KI, 128), jnp.int32),
        pltpu.VMEM((2, KI, 128), jnp.int32),
        pltpu.VMEM((BLK, OUT), jnp.float32),
        pltpu.VMEM((BLK, OUT), jnp.float32),
        pltpu.VMEM((ZR, OUT), jnp.float32),
        pltpu.VMEM_SHARED((N_PAD, OUT), jnp.float32),
        pltpu.VMEM((L,), jnp.int32),
        pltpu.SemaphoreType.DMA,
        pltpu.SemaphoreType.DMA,
        pltpu.SemaphoreType.DMA,
        pltpu.SemaphoreType.DMA,
    ],
    compiler_params=_sc_params,
)(_edge_body)


def _rs(d):
    return lax.rsqrt(jnp.maximum(d, 1.0))


def _mm1_body(x_ref, d_ref, w1_ref, out_ref):
    xs = x_ref[...] * _rs(d_ref[:, 0:1])
    h = jnp.dot(xs, w1_ref[...], preferred_element_type=jnp.float32)
    out_ref[0] = h[:, :OUT]
    out_ref[1] = h[:, OUT:]


def _mm2_body(agg_ref, d_ref, w2_ref, b1_ref, out_ref):
    a = jnp.concatenate([agg_ref[0], agg_ref[1]], axis=1)
    t = (jnp.maximum(a * _rs(d_ref[:, 1:2]) + b1_ref[...], 0.0)
         * _rs(d_ref[:, 0:1]))
    h2 = jnp.dot(t, w2_ref[...], preferred_element_type=jnp.float32)
    out_ref[0] = h2


def _fin_body(p_ref, d_ref, b2_ref, out_ref):
    p = p_ref[0] + p_ref[1]
    out_ref[...] = p * _rs(d_ref[:, 1:2]) + b2_ref[...]


def _dcol_spec():
    return pl.BlockSpec((BN, 2), lambda i: (i, 0))


def _mm1(x, dcol, W1):
    return pl.pallas_call(
        _mm1_body,
        grid=(GRID_N,),
        in_specs=[
            pl.BlockSpec((BN, D_IN), lambda i: (i, 0)),
            _dcol_spec(),
            pl.BlockSpec((D_IN, HID), lambda i: (0, 0)),
        ],
        out_specs=pl.BlockSpec((NC, BN, OUT), lambda i: (0, i, 0)),
        out_shape=jax.ShapeDtypeStruct((NC, N_PAD, OUT), jnp.float32),
    )(x, dcol, W1)


def _mm2(agg, dcol, W2, b1):
    return pl.pallas_call(
        _mm2_body,
        grid=(GRID_N,),
        in_specs=[
            pl.BlockSpec((NC, BN, OUT), lambda i: (0, i, 0)),
            _dcol_spec(),
            pl.BlockSpec((HID, OUT), lambda i: (0, 0)),
            pl.BlockSpec((1, HID), lambda i: (0, 0)),
        ],
        out_specs=pl.BlockSpec((NC, BN, OUT), lambda i: (0, i, 0)),
        out_shape=jax.ShapeDtypeStruct((NC, N_PAD, OUT), jnp.float32),
    )(agg, dcol, W2, b1)


def _fin(p, dcol, b2):
    return pl.pallas_call(
        _fin_body,
        grid=(GRID_N,),
        in_specs=[
            pl.BlockSpec((NC, BN, OUT), lambda i: (0, i, 0)),
            _dcol_spec(),
            pl.BlockSpec((1, OUT), lambda i: (0, 0)),
        ],
        out_specs=pl.BlockSpec((BN, OUT), lambda i: (i, 0)),
        out_shape=jax.ShapeDtypeStruct((N, OUT), jnp.float32),
    )(p, dcol, b2)


def kernel(node_features, edge_index, W1, b1, W2, b2):
    src = edge_index[0].astype(jnp.int32)
    dst = edge_index[1].astype(jnp.int32)
    pad = jnp.full((E_PAD - E,), N, jnp.int32)
    s2 = jnp.concatenate([src, pad]).reshape(R, 128)
    d2 = jnp.concatenate([dst, pad]).reshape(R, 128)

    _, deg = _deg_kernel(s2, d2)
    dcol = jnp.stack([deg[0], deg[1]], axis=1)

    table0 = _mm1(node_features, dcol, W1).reshape(NC * N_PAD, OUT)

    w1r = b1.reshape(1, HID)

    params = jnp.zeros((2, L), jnp.int32)
    params = params.at[0, 0].set(R // NS)
    params = params.at[0, 2].set(N_PAD)
    params = params.at[1, 0].set(R // (NC * NS))
    params = params.at[1, 1].set(R // NC)

    first = jnp.array([True, False])

    def _phase(agg_prev, xs):
        par, is_first = xs
        table = lax.cond(
            is_first,
            lambda a: table0,
            lambda a: _mm2(a, dcol, W2, w1r).reshape(NC * N_PAD, OUT),
            agg_prev)
        agg = _edge_kernel(table, s2, d2, par)
        return agg, None

    agg2, _ = lax.scan(
        _phase, jnp.zeros((NC, N_PAD, OUT), jnp.float32),
        (params, first), length=2)
    return _fin(agg2, dcol, b2.reshape(1, OUT))


# speedup vs baseline: 11.4526x; 1.0680x over previous
import functools

import jax
import jax.numpy as jnp
from jax import lax
from jax.experimental import pallas as pl
from jax.experimental.pallas import tpu as pltpu
from jax.experimental.pallas import tpu_sc as plsc

N = 50000
E = 800000
D_IN = 128
HID = 64
OUT = 32

L = 16
NC = 2
NS = 16

N_PAD = 50176
E_PAD = 802816
R = E_PAD // 128
STRIPE = N_PAD // NS
BN = 1792
GRID_N = N_PAD // BN

_mesh = plsc.VectorSubcoreMesh(core_axis_name="c", subcore_axis_name="s")
_sc_params = pltpu.CompilerParams(use_tc_tiling_on_sc=False,
                                  needs_layout_passes=False)


SB = 28
NSB = (R // NS) // SB


def _deg_body(si_h, di_h, part_h, deg_h, hist, idx0, idx1, acc, tmp0, tmp1,
              is0, is1, rs0, rs1):
    c = lax.axis_index("c")
    s = lax.axis_index("s")

    zeros16 = jnp.zeros((L,), jnp.float32)

    def _zero(i, _):
        hist[pl.ds(i * L, L)] = zeros16
        return ()

    lax.fori_loop(0, N_PAD // L, _zero, (), unroll=4)

    base = s * (R // NS)
    ones16 = jnp.full((L,), 1.0, jnp.float32)

    def _fetch_idx(rows, buf, sem):
        @pl.when(c == 0)
        def _():
            pltpu.async_copy(si_h.at[rows], buf, sem)

        @pl.when(c == 1)
        def _():
            pltpu.async_copy(di_h.at[rows], buf, sem)

    def _drain_idx(sem, buf):
        pltpu.make_async_copy(si_h.at[pl.ds(0, SB)], buf, sem).wait()

    def _scatter_sb(buf):
        def _rows(j, _):
            for g in range(128 // L):
                iv = buf[j, pl.ds(g * L, L)]
                plsc.addupdate_scatter(hist, [iv], ones16)
            return ()

        lax.fori_loop(0, SB, _rows, (), unroll=4)

    _fetch_idx(pl.ds(base, SB), idx0, is0)

    def _sp(p, _):
        r0 = base + 2 * p * SB
        _fetch_idx(pl.ds(r0 + SB, SB), idx1, is1)
        _drain_idx(is0, idx0)
        _scatter_sb(idx0)

        @pl.when(p + 1 < NSB // 2)
        def _():
            _fetch_idx(pl.ds(r0 + 2 * SB, SB), idx0, is0)

        _drain_idx(is1, idx1)
        _scatter_sb(idx1)
        return ()

    lax.fori_loop(0, NSB // 2, _sp, ())

    pltpu.sync_copy(hist, part_h.at[c, s])
    plsc.subcore_barrier()

    sl = pl.ds(s * STRIPE, STRIPE)
    pltpu.sync_copy(part_h.at[c, 0, sl], acc)
    pltpu.async_copy(part_h.at[c, 1, sl], tmp0, rs0)

    def _drain_part(sem, buf):
        pltpu.make_async_copy(part_h.at[c, 0, sl], buf, sem).wait()

    for t in range(1, NS):
        buf, sem = (tmp0, rs0) if t % 2 == 1 else (tmp1, rs1)
        nbuf, nsem = (tmp1, rs1) if t % 2 == 1 else (tmp0, rs0)
        if t + 1 < NS:
            pltpu.async_copy(part_h.at[c, t + 1, sl], nbuf, nsem)
        _drain_part(sem, buf)

        def _add(k, _):
            ksl = pl.ds(k * L, L)
            acc[ksl] = acc[ksl] + buf[ksl]
            return ()

        lax.fori_loop(0, STRIPE // L, _add, (), unroll=4)

    pltpu.sync_copy(acc, deg_h.at[c, sl])


_deg_kernel = functools.partial(
    pl.kernel,
    out_type=(
        jax.ShapeDtypeStruct((NC, NS, N_PAD), jnp.float32),
        jax.ShapeDtypeStruct((NC, N_PAD), jnp.float32),
    ),
    mesh=_mesh,
    scratch_types=[
        pltpu.VMEM((N_PAD,), jnp.float32),
        pltpu.VMEM((SB, 128), jnp.int32),
        pltpu.VMEM((SB, 128), jnp.int32),
        pltpu.VMEM((STRIPE,), jnp.float32),
        pltpu.VMEM((STRIPE,), jnp.float32),
        pltpu.VMEM((STRIPE,), jnp.float32),
        pltpu.SemaphoreType.DMA,
        pltpu.SemaphoreType.DMA,
        pltpu.SemaphoreType.DMA,
        pltpu.SemaphoreType.DMA,
    ],
    compiler_params=_sc_params,
)(_deg_body)


KI = 2
BLK = KI * 128
NBLK = (R // NS) // KI
NPAIR = NBLK // 2
ZR = 196


def _edge_body(rpt, cstr, goff, tab_h, si_h, di_h, out_h, idx_s, idx_d,
               rows0, rows1, zbuf, sh_acc, gs0, gs1, ss0, ss1):
    c = lax.axis_index("c")
    s = lax.axis_index("s")

    zeros16 = jnp.zeros((L,), jnp.float32)

    def _zero(i, _):
        for g in range(OUT // L):
            zbuf[i, pl.ds(g * L, L)] = zeros16
        return ()

    lax.fori_loop(0, zbuf.shape[0], _zero, (), unroll=4)

    zrows = zbuf.shape[0]
    for t in range(STRIPE // zrows):
        pltpu.sync_copy(zbuf, sh_acc.at[pl.ds(s * STRIPE + t * zrows, zrows)])
    plsc.subcore_barrier()

    base = c * cstr + s * rpt
    npair = rpt // (2 * KI)

    def _load_idx(b, buf):
        r0 = base + b * KI
        pltpu.sync_copy(si_h.at[pl.ds(r0, KI)], idx_s.at[buf])
        pltpu.sync_copy(di_h.at[pl.ds(r0, KI)], idx_d.at[buf])
        if goff:
            off = (c * goff).astype(jnp.int32)
            for j in range(KI):
                for g in range(128 // L):
                    gsl = pl.ds(g * L, L)
                    idx_s[buf, j, gsl] = idx_s[buf, j, gsl] + off

    def _fire_gathers(buf, rbuf, gsem):
        for j in range(KI):
            pltpu.async_copy(tab_h.at[idx_s.at[buf, j]],
                             rbuf.at[pl.ds(j * 128, 128)], gsem)

    def _fire_scatters(buf, rbuf, ssem):
        for j in range(KI):
            pltpu.async_copy(rbuf.at[pl.ds(j * 128, 128)],
                             sh_acc.at[idx_d.at[buf, j]], ssem, add=True)

    def _drain(sem, rbuf):
        pltpu.make_async_copy(tab_h.at[pl.ds(0, BLK)], rbuf, sem).wait()

    _load_idx(0, 0)
    _fire_gathers(0, rows0, gs0)

    def _pair(p, _):
        b0 = 2 * p
        _load_idx(b0 + 1, 1)

        @pl.when(p >= 1)
        def _():
            _drain(ss1, rows1)

        _fire_gathers(1, rows1, gs1)

        _drain(gs0, rows0)
        _fire_scatters(0, rows0, ss0)

        @pl.when(p + 1 < npair)
        def _():
            _load_idx(b0 + 2, 0)
            _drain(ss0, rows0)
            _fire_gathers(0, rows0, gs0)

        _drain(gs1, rows1)
        _fire_scatters(1, rows1, ss1)
        return ()

    lax.fori_loop(0, npair, _pair, ())
    _drain(ss0, rows0)
    _drain(ss1, rows1)
    plsc.subcore_barrier()

    sl = pl.ds(s * STRIPE, STRIPE)
    pltpu.sync_copy(sh_acc.at[sl], out_h.at[c, sl])


def _make_edge_kernel(rpt, cstr, goff):
    return functools.partial(
        pl.kernel,
        out_type=jax.ShapeDtypeStruct((NC, N_PAD, OUT), jnp.float32),
        mesh=_mesh,
        scratch_types=[
            pltpu.VMEM((2, KI, 128), jnp.int32),
            pltpu.VMEM((2, KI, 128), jnp.int32),
            pltpu.VMEM((BLK, OUT), jnp.float32),
            pltpu.VMEM((BLK, OUT), jnp.float32),
            pltpu.VMEM((ZR, OUT), jnp.float32),
            pltpu.VMEM_SHARED((N_PAD, OUT), jnp.float32),
            pltpu.SemaphoreType.DMA,
            pltpu.SemaphoreType.DMA,
            pltpu.SemaphoreType.DMA,
            pltpu.SemaphoreType.DMA,
        ],
        compiler_params=_sc_params,
    )(functools.partial(_edge_body, rpt, cstr, goff))


_edge_p0 = _make_edge_kernel(R // NS, 0, N_PAD)
_edge_p1 = _make_edge_kernel(R // (NC * NS), R // NC, 0)


def _rs(d):
    return lax.rsqrt(jnp.maximum(d, 1.0))


def _mm1_body(x_ref, d_ref, w1_ref, out_ref):
    xs = x_ref[...] * _rs(d_ref[:, 0:1])
    h = jnp.dot(xs, w1_ref[...], preferred_element_type=jnp.float32)
    out_ref[0] = h[:, :OUT]
    out_ref[1] = h[:, OUT:]


def _mm2_body(agg_ref, d_ref, w2_ref, b1_ref, out_ref):
    a = jnp.concatenate([agg_ref[0], agg_ref[1]], axis=1)
    t = (jnp.maximum(a * _rs(d_ref[:, 1:2]) + b1_ref[...], 0.0)
         * _rs(d_ref[:, 0:1]))
    h2 = jnp.dot(t, w2_ref[...], preferred_element_type=jnp.float32)
    out_ref[0] = h2


def _fin_body(p_ref, d_ref, b2_ref, out_ref):
    p = p_ref[0] + p_ref[1]
    out_ref[...] = p * _rs(d_ref[:, 1:2]) + b2_ref[...]


def _dcol_spec():
    return pl.BlockSpec((BN, 2), lambda i: (i, 0))


def _mm1(x, dcol, W1):
    return pl.pallas_call(
        _mm1_body,
        grid=(GRID_N,),
        in_specs=[
            pl.BlockSpec((BN, D_IN), lambda i: (i, 0)),
            _dcol_spec(),
            pl.BlockSpec((D_IN, HID), lambda i: (0, 0)),
        ],
        out_specs=pl.BlockSpec((NC, BN, OUT), lambda i: (0, i, 0)),
        out_shape=jax.ShapeDtypeStruct((NC, N_PAD, OUT), jnp.float32),
    )(x, dcol, W1)


def _mm2(agg, dcol, W2, b1):
    return pl.pallas_call(
        _mm2_body,
        grid=(GRID_N,),
        in_specs=[
            pl.BlockSpec((NC, BN, OUT), lambda i: (0, i, 0)),
            _dcol_spec(),
            pl.BlockSpec((HID, OUT), lambda i: (0, 0)),
            pl.BlockSpec((1, HID), lambda i: (0, 0)),
        ],
        out_specs=pl.BlockSpec((NC, BN, OUT), lambda i: (0, i, 0)),
        out_shape=jax.ShapeDtypeStruct((NC, N_PAD, OUT), jnp.float32),
    )(agg, dcol, W2, b1)


def _fin(p, dcol, b2):
    return pl.pallas_call(
        _fin_body,
        grid=(GRID_N,),
        in_specs=[
            pl.BlockSpec((NC, BN, OUT), lambda i: (0, i, 0)),
            _dcol_spec(),
            pl.BlockSpec((1, OUT), lambda i: (0, 0)),
        ],
        out_specs=pl.BlockSpec((BN, OUT), lambda i: (i, 0)),
        out_shape=jax.ShapeDtypeStruct((N, OUT), jnp.float32),
    )(p, dcol, b2)


def kernel(node_features, edge_index, W1, b1, W2, b2):
    src = edge_index[0].astype(jnp.int32)
    dst = edge_index[1].astype(jnp.int32)
    pad = jnp.full((E_PAD - E,), N, jnp.int32)
    s2 = jnp.concatenate([src, pad]).reshape(R, 128)
    d2 = jnp.concatenate([dst, pad]).reshape(R, 128)

    _, deg = _deg_kernel(s2, d2)
    dcol = jnp.stack([deg[0], deg[1]], axis=1)

    table0 = _mm1(node_features, dcol, W1).reshape(NC * N_PAD, OUT)

    w1r = b1.reshape(1, HID)

    pred = s2[0, 0] >= 0

    def _skip(_t):
        return jnp.zeros((NC, N_PAD, OUT), jnp.float32)

    agg1 = lax.cond(pred, lambda t: _edge_p0(t, s2, d2), _skip, table0)
    table1 = _mm2(agg1, dcol, W2, w1r).reshape(NC * N_PAD, OUT)
    agg2 = lax.cond(pred, lambda t: _edge_p1(t, s2, d2), _skip, table1)
    return _fin(agg2, dcol, b2.reshape(1, OUT))
